# baseline (speedup 1.0000x reference)
import functools

import jax
import jax.numpy as jnp
from jax import lax
from jax.experimental import pallas as pl
from jax.experimental.pallas import tpu as pltpu
from jax.experimental.pallas import tpu_sc as plsc

VOCAB = 1000000
EMB = 64
BATCH = 16384
NEG = 10

NC = 2
NS = 16
NW = NC * NS
VSLICE = 31248
SLAB = VSLICE + 64
NITEMS = BATCH * (1 + NEG)
BANK = 6400
SCAN_CH = 4096


def _sc_scores_body(pos_hbm, neg_hbm, vt_hbm, out_hbm,
                    score_out, tag_out,
                    sidx, l_woff, l_b, l_sc, acc,
                    slab0, slab1, vd0, vd1, sem0, sem1):
    wid = lax.axis_index("s") * NC + lax.axis_index("c")
    slabs = (slab0, slab1)
    vds = (vd0, vd1)
    sems = (sem0, sem1)
    iota16 = lax.broadcasted_iota(jnp.int32, (16,), 0)
    lo = wid * VSLICE
    limit = jnp.where(wid == NW - 1, SLAB, VSLICE)

    def init(g, _):
        sl = pl.ds(g * 16, 16)
        l_woff[sl] = jnp.zeros((16,), jnp.int32)
        l_b[sl] = jnp.zeros((16,), jnp.int32)
        l_sc[sl] = jnp.full((16,), -1, jnp.int32)
        acc[sl] = jnp.zeros((16,), jnp.float32)
        return 0
    lax.fori_loop(0, BANK // 16, init, 0, unroll=8)

    def scan_chunk(idx_hbm, chunk, kbase, is_neg, cnt):
        pltpu.sync_copy(idx_hbm.at[pl.ds(chunk * SCAN_CH, SCAN_CH)], sidx)

        def body(i, cnt):
            x = sidx[pl.ds(i * 16, 16)]
            kv = kbase + chunk * SCAN_CH + i * 16 + iota16
            w = x - lo
            m = (w >= 0) & (w < limit)
            b = kv // NEG if is_neg else kv
            sc = BATCH + kv if is_neg else kv
            plsc.store_compressed(l_woff.at[pl.ds(cnt, 16)], w, mask=m)
            plsc.store_compressed(l_b.at[pl.ds(cnt, 16)], b, mask=m)
            plsc.store_compressed(l_sc.at[pl.ds(cnt, 16)], sc, mask=m)
            return cnt + plsc.all_reduce_population_count(m)[0]
        return lax.fori_loop(0, SCAN_CH // 16, body, cnt, unroll=2)

    cnt = 0
    for c in range(BATCH // SCAN_CH):
        cnt = scan_chunk(pos_hbm, c, 0, False, cnt)
    for c in range(BATCH * NEG // SCAN_CH):
        cnt = scan_chunk(neg_hbm, c, 0, True, cnt)
    m_groups = (cnt + 15) // 16

    def issue(d, s):
        pltpu.async_copy(out_hbm.at[pl.ds(d * VOCAB + lo, SLAB)],
                         slabs[s], sems[s])
        pltpu.async_copy(vt_hbm.at[pl.ds(d * BATCH, BATCH)], vds[s], sems[s])

    def drain(s):
        pltpu.make_async_copy(out_hbm.at[pl.ds(0, SLAB)],
                              slabs[s], sems[s]).wait()
        pltpu.make_async_copy(out_hbm.at[pl.ds(0, BATCH)],
                              vds[s], sems[s]).wait()

    def compute(s):
        def body(i, _):
            sl = pl.ds(i * 16, 16)
            u = plsc.load_gather(slabs[s], [l_woff[sl]])
            v = plsc.load_gather(vds[s], [l_b[sl]])
            acc[sl] = acc[sl] + u * v
            return 0
        lax.fori_loop(0, m_groups, body, 0)

    issue(0, 0)

    def step(td, _):
        issue(2 * td + 1, 1)
        drain(0)
        compute(0)

        @pl.when(td < EMB // 2 - 1)
        def _():
            issue(2 * td + 2, 0)
        drain(1)
        compute(1)
        return 0
    lax.fori_loop(0, EMB // 2, step, 0)

    pltpu.sync_copy(acc, score_out.at[pl.ds(wid * BANK, BANK)])
    pltpu.sync_copy(l_sc, tag_out.at[pl.ds(wid * BANK, BANK)])


def _loss_body(s_ref, t_ref, out_ref):
    s = s_ref[...]
    t = t_ref[...]
    valid = t >= 0
    is_pos = t < BATCH
    x = jnp.where(is_pos, s, -s)
    ls = jnp.minimum(x, 0.0) - jnp.log1p(jnp.exp(-jnp.abs(x)))
    w = jnp.where(is_pos, 1.0 / BATCH, 1.0 / (BATCH * NEG))
    out_ref[0, 0] = -jnp.sum(jnp.where(valid, ls * w, 0.0))


@jax.jit
def kernel(in_embed, out_embed, center, pos, neg):
    center = center.astype(jnp.int32)
    pos = pos.astype(jnp.int32)
    neg_flat = jnp.reshape(neg.astype(jnp.int32), (BATCH * NEG,))
    v = jnp.take(in_embed, center, axis=0)
    vt_flat = jnp.reshape(v.T, (EMB * BATCH,))
    out_flat = jnp.reshape(out_embed.T, (VOCAB * EMB,))

    mesh = plsc.VectorSubcoreMesh(core_axis_name="c", subcore_axis_name="s")
    sc_scores = functools.partial(
        pl.kernel,
        mesh=mesh,
        compiler_params=pltpu.CompilerParams(
            needs_layout_passes=False, use_tc_tiling_on_sc=False),
        out_type=[jax.ShapeDtypeStruct((NW * BANK,), jnp.float32),
                  jax.ShapeDtypeStruct((NW * BANK,), jnp.int32)],
        scratch_types=[
            pltpu.VMEM((SCAN_CH,), jnp.int32),
            pltpu.VMEM((BANK,), jnp.int32),
            pltpu.VMEM((BANK,), jnp.int32),
            pltpu.VMEM((BANK,), jnp.int32),
            pltpu.VMEM((BANK,), jnp.float32),
            pltpu.VMEM((SLAB,), jnp.float32),
            pltpu.VMEM((SLAB,), jnp.float32),
            pltpu.VMEM((BATCH,), jnp.float32),
            pltpu.VMEM((BATCH,), jnp.float32),
            pltpu.SemaphoreType.DMA,
            pltpu.SemaphoreType.DMA,
        ],
    )(_sc_scores_body)
    scores, tags = sc_scores(pos, neg_flat, vt_flat, out_flat)

    loss = pl.pallas_call(
        _loss_body,
        out_shape=jax.ShapeDtypeStruct((1, 1), jnp.float32),
        out_specs=pl.BlockSpec(memory_space=pltpu.SMEM),
    )(jnp.reshape(scores, (NW * BANK // 128, 128)),
      jnp.reshape(tags, (NW * BANK // 128, 128)))
    return loss[0, 0]
